# Initial kernel scaffold; baseline (speedup 1.0000x reference)
import jax
import jax.numpy as jnp
from jax.experimental import pallas as pl


def kernel(x, edge_index, edge_attr, current_conditions, target_conditions, params):
    raise NotImplementedError("write your pallas kernel here")


# speedup vs baseline: 2.5191x; 2.5191x over previous
import functools

import jax
import jax.numpy as jnp
from jax import lax
from jax.experimental import pallas as pl
from jax.experimental.pallas import tpu as pltpu
from jax.experimental.pallas import tpu_sc as plsc

HD = 32
CED = 4
_NB = 1000
_EB = 1024
_F32 = jnp.float32


def _fullspec(a):
    r = a.ndim
    return pl.BlockSpec(a.shape, lambda i, _r=r: (0,) * _r)


def _rowspec(rows, cols):
    return pl.BlockSpec((rows, cols), lambda i: (i, 0))


def _ln(h, g, b):
    m = jnp.mean(h, axis=-1, keepdims=True)
    v = jnp.mean((h - m) ** 2, axis=-1, keepdims=True)
    return (h - m) * lax.rsqrt(v + 1e-5) * g + b


def _dot(a, b):
    return jnp.dot(a, b, preferred_element_type=_F32)


def _enc_flat(p):
    return [p["W1"], p["b1"].reshape(1, -1), p["W2"], p["b2"].reshape(1, -1),
            p["g"].reshape(1, -1), p["be"].reshape(1, -1)]


def _enc_apply(inp, W1, b1, W2, b2, g, be):
    h = jnp.maximum(_dot(inp, W1[...]) + b1[...], 0.0)
    h = _dot(h, W2[...]) + b2[...]
    return _ln(h, g[...], be[...])


def _encoder_stage(x, cc, tc, params, n, ci1p):
    weights = []
    for name in ("cve", "cte", "tve", "tte", "time", "fusion"):
        weights += _enc_flat(params[name])

    def body(x_r, cc_r, tc_r, *rest):
        wr = rest[:36]
        cf_o, ex_o = rest[36:]
        cv = _enc_apply(cc_r[:, 0:1], *wr[0:6])
        ct = _enc_apply(cc_r[:, 1:2], *wr[6:12])
        tv = _enc_apply(tc_r[:, 0:1], *wr[12:18])
        tt = _enc_apply(tc_r[:, 1:2], *wr[18:24])
        tm = _enc_apply(jnp.concatenate([cc_r[:, 2:3], tc_r[:, 2:3]], axis=1),
                        *wr[24:30])
        cf = jnp.concatenate([cv, ct, tv, tt, tm], axis=1)
        fused = _enc_apply(cf, *wr[30:36])
        cf_o[...] = cf
        pad = jnp.zeros((cf.shape[0], ci1p - 3 - HD), _F32)
        ex_o[...] = jnp.concatenate([x_r[...], fused, pad], axis=1)

    grid = (n // _NB,)
    in_specs = [_rowspec(_NB, 3), _rowspec(_NB, 3), _rowspec(_NB, 3)]
    in_specs += [_fullspec(w) for w in weights]
    return pl.pallas_call(
        body,
        grid=grid,
        in_specs=in_specs,
        out_specs=[_rowspec(_NB, 5 * CED), _rowspec(_NB, ci1p)],
        out_shape=[jax.ShapeDtypeStruct((n, 5 * CED), _F32),
                   jax.ShapeDtypeStruct((n, ci1p), _F32)],
    )(x, cc, tc, *weights)


def _sc_gather(table, idx2d, ep):
    d = table.shape[1]
    pw = ep // 32
    nch = pw // 128
    tb = 1024
    nout = pw // tb
    mesh = plsc.VectorSubcoreMesh(core_axis_name="c", subcore_axis_name="s",
                                   num_cores=2, num_subcores=16)

    @functools.partial(
        pl.kernel,
        out_type=jax.ShapeDtypeStruct((ep, d), _F32),
        mesh=mesh,
        compiler_params=pltpu.CompilerParams(use_tc_tiling_on_sc=False),
        scratch_types=[pltpu.VMEM((nch, 128), jnp.int32),
                       pltpu.VMEM((tb, d), _F32),
                       pltpu.SemaphoreType.DMA],
    )
    def k(table_hbm, idx_hbm, out_hbm, idx_v, rows_v, sem):
        w = lax.axis_index("s") * 2 + lax.axis_index("c")
        pltpu.sync_copy(idx_hbm.at[pl.ds(w * nch, nch)], idx_v)

        def outer(t, carry):
            cps = [
                pltpu.async_copy(table_hbm.at[idx_v.at[t * (tb // 128) + g]],
                                 rows_v.at[pl.ds(g * 128, 128)], sem)
                for g in range(tb // 128)
            ]
            for cp in cps:
                cp.wait()
            pltpu.sync_copy(rows_v, out_hbm.at[pl.ds(w * pw + t * tb, tb)])
            return carry

        lax.fori_loop(0, nout, outer, 0)

    return k(table, idx2d)


def _sc_scatter(msg, idx2d, z32, ep, npad, with_counts, zc=None, ones=None):
    pw = ep // 32
    nch = pw // 128
    tb = 1024
    nout = pw // tb
    pt = npad // 16
    mesh = plsc.VectorSubcoreMesh(core_axis_name="c", subcore_axis_name="s",
                                   num_cores=2, num_subcores=16)

    out_type = [jax.ShapeDtypeStruct((2 * npad, 32), _F32)]
    scratch = [pltpu.VMEM((nch, 128), jnp.int32),
               pltpu.VMEM((tb, 32), _F32),
               pltpu.VMEM_SHARED((npad, 32), _F32)]
    if with_counts:
        out_type.append(jax.ShapeDtypeStruct((2 * npad, 16), _F32))
        scratch += [pltpu.VMEM((128, 16), _F32),
                    pltpu.VMEM((pt, 16), _F32),
                    pltpu.VMEM_SHARED((npad, 16), _F32)]

    @functools.partial(pl.kernel, out_type=out_type, mesh=mesh,
                       compiler_params=pltpu.CompilerParams(
                           use_tc_tiling_on_sc=False),
                       scratch_types=scratch)
    def k(*refs):
        if with_counts:
            (msg_hbm, idx_hbm, z_hbm, zc_hbm, ones_hbm, p_hbm, pc_hbm,
             idx_v, rows_v, acc_sh, ones_v, cbuf, cnt_sh) = refs
        else:
            msg_hbm, idx_hbm, z_hbm, p_hbm, idx_v, rows_v, acc_sh = refs
        c = lax.axis_index("c")
        s = lax.axis_index("s")
        w = s * 2 + c

        pltpu.sync_copy(z_hbm.at[pl.ds(s * pt, pt)], rows_v.at[pl.ds(0, pt)])
        pltpu.sync_copy(rows_v.at[pl.ds(0, pt)], acc_sh.at[pl.ds(s * pt, pt)])
        if with_counts:
            pltpu.sync_copy(zc_hbm.at[pl.ds(s * pt, pt)], cbuf)
            pltpu.sync_copy(cbuf, cnt_sh.at[pl.ds(s * pt, pt)])
            pltpu.sync_copy(ones_hbm, ones_v)
        plsc.subcore_barrier()

        pltpu.sync_copy(idx_hbm.at[pl.ds(w * nch, nch)], idx_v)

        def outer(t, carry):
            pltpu.sync_copy(msg_hbm.at[pl.ds(w * pw + t * tb, tb)], rows_v)
            for g in range(tb // 128):
                row = idx_v.at[t * (tb // 128) + g]
                pltpu.sync_copy(rows_v.at[pl.ds(g * 128, 128)],
                                acc_sh.at[row], add=True)
                if with_counts:
                    pltpu.sync_copy(ones_v, cnt_sh.at[row], add=True)
            return carry

        lax.fori_loop(0, nout, outer, 0)
        plsc.subcore_barrier()

        pltpu.sync_copy(acc_sh.at[pl.ds(s * pt, pt)], rows_v.at[pl.ds(0, pt)])
        pltpu.sync_copy(rows_v.at[pl.ds(0, pt)],
                        p_hbm.at[pl.ds(c * npad + s * pt, pt)])
        if with_counts:
            pltpu.sync_copy(cnt_sh.at[pl.ds(s * pt, pt)], cbuf)
            pltpu.sync_copy(cbuf, pc_hbm.at[pl.ds(c * npad + s * pt, pt)])

    if with_counts:
        return tuple(k(msg, idx2d, z32, zc, ones))
    return tuple(k(msg, idx2d, z32))


def _msg_stage(ea, xs, a2p, r_mat, s_mat, bp, nw1, nb1, ep):
    d = xs.shape[1]

    def body(ea_r, xs_r, a2_r, r_r, s_r, b_r, w1_r, b1_r, out_r):
        h = jnp.maximum(ea_r[...] * w1_r[...] + b1_r[...], 0.0)
        p = _dot(xs_r[...], a2_r[...])
        hr = _dot(h, r_r[...])
        out_r[...] = _dot(p * hr, s_r[...]) + _dot(xs_r[...], b_r[...])

    return pl.pallas_call(
        body,
        grid=(ep // _EB,),
        in_specs=[_rowspec(_EB, 1), _rowspec(_EB, d), _fullspec(a2p),
                  _fullspec(r_mat), _fullspec(s_mat), _fullspec(bp),
                  _fullspec(nw1), _fullspec(nb1)],
        out_specs=_rowspec(_EB, HD),
        out_shape=jax.ShapeDtypeStruct((ep, HD), _F32),
    )(ea, xs, a2p, r_mat, s_mat, bp, nw1, nb1)


def _aggr(p0, p1, c0, c1):
    cnt = jnp.maximum(c0[...] + c1[...], 1.0)
    return (p0[...] + p1[...]) / cnt


def _ep_first(p0, p1, c0, c1, ex, root, bias, g, be, n):
    def body(p0_r, p1_r, c0_r, c1_r, ex_r, rt_r, b_r, g_r, be_r, out_r):
        o = _aggr(p0_r, p1_r, c0_r, c1_r) + _dot(ex_r[...], rt_r[...]) + b_r[...]
        out_r[...] = jnp.maximum(_ln(o, g_r[...], be_r[...]), 0.0)

    d = ex.shape[1]
    return pl.pallas_call(
        body,
        grid=(n // _NB,),
        in_specs=[_rowspec(_NB, HD), _rowspec(_NB, HD), _rowspec(_NB, 1),
                  _rowspec(_NB, 1), _rowspec(_NB, d), _fullspec(root),
                  _fullspec(bias), _fullspec(g), _fullspec(be)],
        out_specs=_rowspec(_NB, HD),
        out_shape=jax.ShapeDtypeStruct((n, HD), _F32),
    )(p0, p1, c0, c1, ex, root, bias, g, be)


def _ep_mid(p0, p1, c0, c1, h, root, bias, g, be, n):
    def body(p0_r, p1_r, c0_r, c1_r, h_r, rt_r, b_r, g_r, be_r, out_r):
        o = _aggr(p0_r, p1_r, c0_r, c1_r) + _dot(h_r[...], rt_r[...]) + b_r[...]
        out_r[...] = jnp.maximum(_ln(o, g_r[...], be_r[...]), 0.0) + h_r[...]

    return pl.pallas_call(
        body,
        grid=(n // _NB,),
        in_specs=[_rowspec(_NB, HD), _rowspec(_NB, HD), _rowspec(_NB, 1),
                  _rowspec(_NB, 1), _rowspec(_NB, HD), _fullspec(root),
                  _fullspec(bias), _fullspec(g), _fullspec(be)],
        out_specs=_rowspec(_NB, HD),
        out_shape=jax.ShapeDtypeStruct((n, HD), _F32),
    )(p0, p1, c0, c1, h, root, bias, g, be)


def _ep_last(p0, p1, c0, c1, h, root, bias, cf, x, pred, n):
    pw = []
    for W, b in pred:
        pw += [W, b.reshape(1, -1)]

    def body(p0_r, p1_r, c0_r, c1_r, h_r, rt_r, b_r, cf_r, x_r, *rest):
        wr = rest[:-1]
        out_r = rest[-1]
        hh = _aggr(p0_r, p1_r, c0_r, c1_r) + _dot(h_r[...], rt_r[...]) + b_r[...]
        d = jnp.concatenate([hh, cf_r[...]], axis=1)
        nlayer = len(wr) // 2
        for i in range(nlayer):
            d = _dot(d, wr[2 * i][...]) + wr[2 * i + 1][...]
            if i < nlayer - 1:
                d = jnp.maximum(d, 0.0)
        out_r[...] = x_r[...] + d

    return pl.pallas_call(
        body,
        grid=(n // _NB,),
        in_specs=[_rowspec(_NB, HD), _rowspec(_NB, HD), _rowspec(_NB, 1),
                  _rowspec(_NB, 1), _rowspec(_NB, HD), _fullspec(root),
                  _fullspec(bias), _rowspec(_NB, 5 * CED), _rowspec(_NB, 3)]
        + [_fullspec(w) for w in pw],
        out_specs=_rowspec(_NB, 3),
        out_shape=jax.ShapeDtypeStruct((n, 3), _F32),
    )(p0, p1, c0, c1, h, root, bias, cf, x, *pw)


def _conv_weights(p, ci, cip):
    co = HD
    a2 = p["nW2"].reshape(HD, ci, co).transpose(1, 0, 2).reshape(ci, HD * co)
    a2p = jnp.zeros((cip, HD * co), _F32).at[:ci].set(a2)
    bp = jnp.zeros((cip, co), _F32).at[:ci].set(p["nb2"].reshape(ci, co))
    rootp = jnp.zeros((cip, co), _F32).at[:ci].set(p["root"])
    return (a2p, bp, rootp, p["nW1"].reshape(1, HD),
            p["nb1"].reshape(1, HD), p["bias"].reshape(1, HD))


def kernel(x, edge_index, edge_attr, current_conditions, target_conditions,
           params):
    n = x.shape[0]
    e = edge_index.shape[1]
    ep = ((e + 32767) // 32768) * 32768
    npad = ((n + 127) // 128) * 128
    ci1 = 3 + HD
    ci1p = 48

    co = HD
    j = jnp.arange(HD * co, dtype=jnp.int32)
    r_mat = (j[None, :] // co == jnp.arange(HD, dtype=jnp.int32)[:, None]
             ).astype(_F32)
    s_mat = (j[:, None] % co == jnp.arange(co, dtype=jnp.int32)[None, :]
             ).astype(_F32)

    src = edge_index[0]
    dst = edge_index[1]
    pad = ep - e
    src2d = jnp.concatenate([src, jnp.zeros((pad,), jnp.int32)]
                            ).reshape(ep // 128, 128)
    dst2d = jnp.concatenate([dst, jnp.full((pad,), n, jnp.int32)]
                            ).reshape(ep // 128, 128)
    ea = jnp.concatenate([edge_attr, jnp.zeros((pad, 1), _F32)])
    z32 = jnp.zeros((npad, 32), _F32)
    zc = jnp.zeros((npad, 16), _F32)
    ones = jnp.ones((128, 16), _F32)

    cf, ex = _encoder_stage(x, current_conditions, target_conditions,
                            params, n, ci1p)

    convs = [
        (params["icv"], ci1, ci1p, ex),
        (params["hcv"][0], HD, HD, None),
        (params["ocv"], HD, HD, None),
    ]

    h = None
    out = None
    c0 = c1 = None
    for li, (cp, ci, cip, table0) in enumerate(convs):
        table = table0 if table0 is not None else h
        a2p, bp, rootp, nw1, nb1, bias = _conv_weights(cp, ci, cip)
        xs = _sc_gather(table, src2d, ep)
        msg = _msg_stage(ea, xs, a2p, r_mat, s_mat, bp, nw1, nb1, ep)
        if li == 0:
            p, pc = _sc_scatter(msg, dst2d, z32, ep, npad, True, zc, ones)
            c0 = pc[:n, 0:1]
            c1 = pc[npad:npad + n, 0:1]
        else:
            (p,) = _sc_scatter(msg, dst2d, z32, ep, npad, False)
        p0 = p[:n]
        p1 = p[npad:npad + n]
        if li == 0:
            g = params["ibn"]["g"].reshape(1, -1)
            be = params["ibn"]["be"].reshape(1, -1)
            h = _ep_first(p0, p1, c0, c1, ex, rootp, bias, g, be, n)
        elif li == 1:
            g = params["hbn"][0]["g"].reshape(1, -1)
            be = params["hbn"][0]["be"].reshape(1, -1)
            h = _ep_mid(p0, p1, c0, c1, h, rootp, bias, g, be, n)
        else:
            out = _ep_last(p0, p1, c0, c1, h, rootp, bias, cf, x,
                           params["pred"], n)
    return out
